# baseline (device time: 8793 ns/iter reference)
import jax
import jax.numpy as jnp
from jax import lax
from jax.experimental import pallas as pl
from jax.experimental.pallas import tpu as pltpu

N_DEV = 4
N_TOK = 256
D_IN = 128
D_OUT = 256
E_PER = 2
CAP = 25
TOK_PER = N_TOK // N_DEV


def kernel(x, router_W, route_idx, expert_W):
    def body(x_ref, rw_ref, idx_ref, w_ref, out_ref,
             pbuf, gbuf, recv_buf, send_sems, recv_sems):
        my = lax.axis_index("i")

        barrier_sem = pltpu.get_barrier_semaphore()
        for off in range(1, N_DEV):
            pl.semaphore_signal(
                barrier_sem, inc=1,
                device_id=((my + off) % N_DEV,),
                device_id_type=pl.DeviceIdType.MESH,
            )
        pl.semaphore_wait(barrier_sem, N_DEV - 1)

        idx = idx_ref[:, :]
        col = lax.broadcasted_iota(jnp.int32, (N_TOK, E_PER), 1)
        local_e = E_PER * my + col
        onehot = (idx == local_e).astype(jnp.float32)
        ri = lax.broadcasted_iota(jnp.int32, (N_TOK, N_TOK), 0)
        ci = lax.broadcasted_iota(jnp.int32, (N_TOK, N_TOK), 1)
        tril = (ri >= ci).astype(jnp.float32)
        cum = jnp.dot(tril, onehot, preferred_element_type=jnp.float32)
        keep = onehot * (cum <= CAP).astype(jnp.float32)

        gbuf[:, :] = keep.astype(jnp.bfloat16)

        def chunk(t):
            rows = pl.ds(t * TOK_PER, TOK_PER)
            xc = x_ref[rows, :].astype(jnp.bfloat16)
            g = gbuf[rows, :]
            acc = jnp.zeros((TOK_PER, D_OUT), jnp.float32)
            for l in range(E_PER):
                w = w_ref[l, :, :].astype(jnp.bfloat16)
                acc = acc + jnp.dot(xc * g[:, l:l + 1], w,
                                    preferred_element_type=jnp.float32)
            return acc

        rdmas = []
        for off in (2, 1, 3):
            t = (my + off) % N_DEV
            pbuf[pl.ds(t * TOK_PER, TOK_PER), :] = chunk(t).astype(jnp.bfloat16)
            rdma = pltpu.make_async_remote_copy(
                src_ref=pbuf.at[pl.ds(t * TOK_PER, TOK_PER), :],
                dst_ref=recv_buf.at[off - 1],
                send_sem=send_sems.at[off - 1],
                recv_sem=recv_sems.at[off - 1],
                device_id=(t,),
                device_id_type=pl.DeviceIdType.MESH,
            )
            rdma.start()
            rdmas.append((off, rdma))

        total = chunk(my)
        for off in (1, 3, 2):
            rdma = dict(rdmas)[off]
            rdma.wait_recv()
            total = total + recv_buf[off - 1, :, :].astype(jnp.float32)
        out_ref[:, :] = total.astype(jnp.bfloat16)
        for _, rdma in rdmas:
            rdma.wait_send()

    return pl.pallas_call(
        body,
        out_shape=jax.ShapeDtypeStruct((TOK_PER, D_OUT), jnp.bfloat16),
        in_specs=[
            pl.BlockSpec(memory_space=pltpu.VMEM),
            pl.BlockSpec(memory_space=pltpu.VMEM),
            pl.BlockSpec(memory_space=pltpu.VMEM),
            pl.BlockSpec(memory_space=pltpu.VMEM),
        ],
        out_specs=pl.BlockSpec(memory_space=pltpu.VMEM),
        scratch_shapes=[
            pltpu.VMEM((N_TOK, D_OUT), jnp.bfloat16),
            pltpu.VMEM((N_TOK, E_PER), jnp.bfloat16),
            pltpu.VMEM((N_DEV - 1, TOK_PER, D_OUT), jnp.bfloat16),
            pltpu.SemaphoreType.DMA((N_DEV - 1,)),
            pltpu.SemaphoreType.DMA((N_DEV - 1,)),
        ],
        compiler_params=pltpu.CompilerParams(collective_id=0),
    )(x, router_W, route_idx, expert_W)


# device time: 8159 ns/iter; 1.0777x vs baseline; 1.0777x over previous
import jax
import jax.numpy as jnp
from jax import lax
from jax.experimental import pallas as pl
from jax.experimental.pallas import tpu as pltpu

N_DEV = 4
N_TOK = 256
D_IN = 128
D_OUT = 256
E_PER = 2
CAP = 25
TOK_PER = N_TOK // N_DEV


def kernel(x, router_W, route_idx, expert_W):
    def body(x_ref, rw_ref, idx_ref, w_ref, out_ref,
             pbuf, gbuf, recv_buf, send_sems, recv_sems):
        my = lax.axis_index("i")

        barrier_sem = pltpu.get_barrier_semaphore()
        for off in range(1, N_DEV):
            pl.semaphore_signal(
                barrier_sem, inc=1,
                device_id=((my + off) % N_DEV,),
                device_id_type=pl.DeviceIdType.MESH,
            )

        idx = idx_ref[:, :]
        col = lax.broadcasted_iota(jnp.int32, (N_TOK, E_PER), 1)
        local_e = E_PER * my + col
        onehot = (idx == local_e).astype(jnp.float32)
        ri = lax.broadcasted_iota(jnp.int32, (N_TOK, N_TOK), 0)
        ci = lax.broadcasted_iota(jnp.int32, (N_TOK, N_TOK), 1)
        tril = (ri >= ci).astype(jnp.float32)
        cum = jnp.dot(tril, onehot, preferred_element_type=jnp.float32)
        keep = onehot * (cum <= CAP).astype(jnp.float32)

        gbuf[:, :] = keep.astype(jnp.bfloat16)

        def chunk(t):
            rows = pl.ds(t * TOK_PER, TOK_PER)
            xc = x_ref[rows, :].astype(jnp.bfloat16)
            g = gbuf[rows, :]
            acc = jnp.zeros((TOK_PER, D_OUT), jnp.float32)
            for l in range(E_PER):
                w = w_ref[l, :, :].astype(jnp.bfloat16)
                acc = acc + jnp.dot(xc * g[:, l:l + 1], w,
                                    preferred_element_type=jnp.float32)
            return acc

        for off in (2, 1, 3):
            t = (my + off) % N_DEV
            pbuf[pl.ds(t * TOK_PER, TOK_PER), :] = chunk(t).astype(jnp.bfloat16)
        total = chunk(my)

        pl.semaphore_wait(barrier_sem, N_DEV - 1)

        rdmas = []
        for off in (2, 1, 3):
            t = (my + off) % N_DEV
            rdma = pltpu.make_async_remote_copy(
                src_ref=pbuf.at[pl.ds(t * TOK_PER, TOK_PER), :],
                dst_ref=recv_buf.at[off - 1],
                send_sem=send_sems.at[off - 1],
                recv_sem=recv_sems.at[off - 1],
                device_id=(t,),
                device_id_type=pl.DeviceIdType.MESH,
            )
            rdma.start()
            rdmas.append((off, rdma))

        for off in (1, 3, 2):
            rdma = dict(rdmas)[off]
            rdma.wait_recv()
            total = total + recv_buf[off - 1, :, :].astype(jnp.float32)
        out_ref[:, :] = total
        for _, rdma in rdmas:
            rdma.wait_send()

    return pl.pallas_call(
        body,
        out_shape=jax.ShapeDtypeStruct((TOK_PER, D_OUT), jnp.float32),
        in_specs=[
            pl.BlockSpec(memory_space=pltpu.VMEM),
            pl.BlockSpec(memory_space=pltpu.VMEM),
            pl.BlockSpec(memory_space=pltpu.VMEM),
            pl.BlockSpec(memory_space=pltpu.VMEM),
        ],
        out_specs=pl.BlockSpec(memory_space=pltpu.VMEM),
        scratch_shapes=[
            pltpu.VMEM((N_TOK, D_OUT), jnp.bfloat16),
            pltpu.VMEM((N_TOK, E_PER), jnp.bfloat16),
            pltpu.VMEM((N_DEV - 1, TOK_PER, D_OUT), jnp.bfloat16),
            pltpu.SemaphoreType.DMA((N_DEV - 1,)),
            pltpu.SemaphoreType.DMA((N_DEV - 1,)),
        ],
        compiler_params=pltpu.CompilerParams(collective_id=0),
    )(x, router_W, route_idx, expert_W)
